# Initial kernel scaffold; baseline (speedup 1.0000x reference)
import functools

import jax
import jax.numpy as jnp
from jax import lax
from jax.experimental import pallas as pl
from jax.experimental.pallas import tpu as pltpu
from jax.experimental.pallas import tpu_sc as plsc

NPX = 512 * 512
NCH = 64 * 3
NW = 32
CPW = NCH // NW
K = 16384
CHUNKS = NPX // K
VPC = K // 16
U = 8


def _histeq_body(x_hbm, out_hbm, in_v, out_v, hist_v, cum_v, lut_v):
    wid = lax.axis_index("s") * 2 + lax.axis_index("c")
    lanes = lax.iota(jnp.int32, 16)
    laneoff = lanes * 256
    lanes_f = lanes.astype(jnp.float32)
    zeros16 = jnp.zeros((16,), jnp.float32)
    ones16 = jnp.ones((16,), jnp.float32)

    def per_channel(ci, carry_unused):
        base = (wid * CPW + ci) * NPX

        def zero_hist(i, c):
            hist_v[pl.ds(i * 16, 16)] = zeros16
            return c

        lax.fori_loop(0, 256, zero_hist, 0)

        def hist_chunk(kc, c):
            pltpu.sync_copy(x_hbm.at[pl.ds(base + kc * K, K)], in_v)

            def grp(g, cc):
                for u in range(U):
                    x = in_v[pl.ds((g * U + u) * 16, 16)]
                    b = jnp.clip(x, 0.0, 255.0).astype(jnp.int32)
                    plsc.addupdate_scatter(hist_v, [laneoff + b], ones16)
                return cc

            lax.fori_loop(0, VPC // U, grp, 0)
            return c

        lax.fori_loop(0, CHUNKS, hist_chunk, 0)

        carry = jnp.float32(0.0)
        for j in range(16):
            v = hist_v[pl.ds(16 * j, 16)]
            for l in range(1, 16):
                v = v + hist_v[pl.ds(l * 256 + 16 * j, 16)]
            c = plsc.cumsum(v) + carry
            cum_v[pl.ds(16 * j, 16)] = c
            carry = cum_v[16 * j + 15]
        total = carry

        sml = jnp.float32(0.0)
        for j in range(16):
            c = cum_v[pl.ds(16 * j, 16)]
            sml = jnp.maximum(sml, jnp.max(jnp.where(c < total, c, 0.0)))

        step = jnp.floor(sml / 255.0)
        off = jnp.floor(step / 2.0)
        den = jnp.maximum(step, 1.0)
        is_id = step == 0.0

        lut_v[pl.ds(0, 16)] = zeros16
        for j in range(16):
            c = cum_v[pl.ds(16 * j, 16)]
            vals = jnp.clip(jnp.floor((c + off) / den), 0.0, 255.0)
            idf = lanes_f + jnp.float32(16 * j + 1)
            vals = jnp.where(is_id, idf, vals)
            plsc.store_scatter(lut_v, [lanes + (16 * j + 1)], vals)

        def remap_chunk(kc, c):
            pltpu.sync_copy(x_hbm.at[pl.ds(base + kc * K, K)], in_v)

            def grp(g, cc):
                for u in range(U):
                    o = (g * U + u) * 16
                    x = in_v[pl.ds(o, 16)]
                    b = jnp.clip(x, 0.0, 255.0).astype(jnp.int32)
                    out_v[pl.ds(o, 16)] = plsc.load_gather(lut_v, [b])
                return cc

            lax.fori_loop(0, VPC // U, grp, 0)
            pltpu.sync_copy(out_v, out_hbm.at[pl.ds(base + kc * K, K)])
            return c

        lax.fori_loop(0, CHUNKS, remap_chunk, 0)
        return carry_unused

    lax.fori_loop(0, CPW, per_channel, 0)


def kernel(pic):
    B, C, H, W = pic.shape
    flat = pic.reshape(B * C * H * W)
    mesh = plsc.VectorSubcoreMesh(core_axis_name="c", subcore_axis_name="s")
    f = pl.kernel(
        _histeq_body,
        out_type=jax.ShapeDtypeStruct((NCH * NPX,), jnp.float32),
        mesh=mesh,
        scratch_types=[
            pltpu.VMEM((K,), jnp.float32),
            pltpu.VMEM((K,), jnp.float32),
            pltpu.VMEM((4096,), jnp.float32),
            pltpu.VMEM((256,), jnp.float32),
            pltpu.VMEM((272,), jnp.float32),
        ],
    )
    out = f(flat)
    return out.reshape(B, C, H, W)


# speedup vs baseline: 307.2991x; 307.2991x over previous
import functools

import jax
import jax.numpy as jnp
from jax import lax
from jax.experimental import pallas as pl
from jax.experimental.pallas import tpu as pltpu
from jax.experimental.pallas import tpu_sc as plsc

NPX = 512 * 512
NCH = 64 * 3
NW = 32
CPW = NCH // NW
K = 16384
CHUNKS = NPX // K
VPC = K // 16
U = 8


def _histeq_body(x_hbm, out_hbm, in_v, out_v, hist_v, cum_v, lut_v):
    wid = lax.axis_index("s") * 2 + lax.axis_index("c")
    lanes = lax.iota(jnp.int32, 16)
    laneoff = lanes * 256
    lanes_f = lanes.astype(jnp.float32)
    zeros16 = jnp.zeros((16,), jnp.float32)
    ones16 = jnp.ones((16,), jnp.float32)

    def per_channel(ci, carry_unused):
        base = (wid * CPW + ci) * NPX

        def zero_hist(i, c):
            hist_v[pl.ds(i * 16, 16)] = zeros16
            return c

        lax.fori_loop(0, 256, zero_hist, 0)

        def hist_chunk(kc, c):
            pltpu.sync_copy(x_hbm.at[pl.ds(base + kc * K, K)], in_v)

            def grp(g, cc):
                for u in range(U):
                    x = in_v[pl.ds((g * U + u) * 16, 16)]
                    b = jnp.clip(x, 0.0, 255.0).astype(jnp.int32)
                    plsc.addupdate_scatter(hist_v, [laneoff + b], ones16)
                return cc

            lax.fori_loop(0, VPC // U, grp, 0)
            return c

        lax.fori_loop(0, CHUNKS, hist_chunk, 0)

        carry = jnp.float32(0.0)
        for j in range(16):
            v = hist_v[pl.ds(16 * j, 16)]
            for l in range(1, 16):
                v = v + hist_v[pl.ds(l * 256 + 16 * j, 16)]
            c = plsc.cumsum(v) + carry
            cum_v[pl.ds(16 * j, 16)] = c
            carry = c[15]
        total = carry

        sml = jnp.float32(0.0)
        for j in range(16):
            c = cum_v[pl.ds(16 * j, 16)]
            sml = jnp.maximum(sml, jnp.max(jnp.where(c < total, c, 0.0)))

        def ffloor(x):
            return x.astype(jnp.int32).astype(jnp.float32)

        step = ffloor(jnp.full((16,), sml, jnp.float32) / 255.0)
        off = ffloor(step * 0.5)
        den = jnp.maximum(step, 1.0)
        is_id = step == 0.0

        lut_v[pl.ds(0, 16)] = zeros16
        for j in range(16):
            c = cum_v[pl.ds(16 * j, 16)]
            vals = jnp.clip(ffloor((c + off) / den), 0.0, 255.0)
            idf = lanes_f + jnp.float32(16 * j + 1)
            vals = jnp.where(is_id, idf, vals)
            plsc.store_scatter(lut_v, [lanes + (16 * j + 1)], vals)

        def remap_chunk(kc, c):
            pltpu.sync_copy(x_hbm.at[pl.ds(base + kc * K, K)], in_v)

            def grp(g, cc):
                for u in range(U):
                    o = (g * U + u) * 16
                    x = in_v[pl.ds(o, 16)]
                    b = jnp.clip(x, 0.0, 255.0).astype(jnp.int32)
                    out_v[pl.ds(o, 16)] = plsc.load_gather(lut_v, [b])
                return cc

            lax.fori_loop(0, VPC // U, grp, 0)
            pltpu.sync_copy(out_v, out_hbm.at[pl.ds(base + kc * K, K)])
            return c

        lax.fori_loop(0, CHUNKS, remap_chunk, 0)
        return carry_unused

    lax.fori_loop(0, CPW, per_channel, 0)


def kernel(pic):
    B, C, H, W = pic.shape
    flat = pic.reshape(B * C * H * W)
    mesh = plsc.VectorSubcoreMesh(core_axis_name="c", subcore_axis_name="s")
    f = pl.kernel(
        _histeq_body,
        out_type=jax.ShapeDtypeStruct((NCH * NPX,), jnp.float32),
        mesh=mesh,
        compiler_params=pltpu.CompilerParams(needs_layout_passes=False),
        scratch_types=[
            pltpu.VMEM((K,), jnp.float32),
            pltpu.VMEM((K,), jnp.float32),
            pltpu.VMEM((4096,), jnp.float32),
            pltpu.VMEM((256,), jnp.float32),
            pltpu.VMEM((272,), jnp.float32),
        ],
    )
    out = f(flat)
    return out.reshape(B, C, H, W)


# speedup vs baseline: 872.7099x; 2.8399x over previous
import functools

import jax
import jax.numpy as jnp
from jax import lax
from jax.experimental import pallas as pl
from jax.experimental.pallas import tpu as pltpu
from jax.experimental.pallas import tpu_sc as plsc

NPX = 512 * 512
NCH = 64 * 3
NW = 32
CPW = NCH // NW
K = 16384
CHUNKS = NPX // K
U = 8


def _histeq_body(x_hbm, out_hbm, in0, in1, out0, out1, hist_v, cum_v, lut_v,
                 sem0, sem1, osem0, osem1):
    wid = lax.axis_index("s") * 2 + lax.axis_index("c")
    lanes = lax.iota(jnp.int32, 16)
    laneoff = lanes * 256
    lanes_f = lanes.astype(jnp.float32)
    zeros16 = jnp.zeros((16,), jnp.float32)
    ones16 = jnp.ones((16,), jnp.float32)
    ins = ((in0, sem0), (in1, sem1))
    outs = ((out0, osem0), (out1, osem1))

    def per_channel(ci, carry_unused):
        base = (wid * CPW + ci) * NPX

        @plsc.parallel_loop(0, 4096, step=16, unroll=U)
        def _(o):
            hist_v[pl.ds(o, 16)] = zeros16

        pltpu.async_copy(x_hbm.at[pl.ds(base, K)], in0, sem0)

        def hist_grp(g, c):
            idx0 = g * 2
            for b, (inb, semb) in enumerate(ins):
                idx = idx0 + b
                pltpu.make_async_copy(x_hbm.at[pl.ds(base, K)], inb, semb).wait()
                nxt = idx + 1
                othb, othsem = ins[1 - b]

                @pl.when(nxt < CHUNKS)
                def _():
                    pltpu.async_copy(
                        x_hbm.at[pl.ds(base + nxt * K, K)], othb, othsem)

                @plsc.parallel_loop(0, K, step=16, unroll=U)
                def _(o):
                    x = inb[pl.ds(o, 16)]
                    bn = x.astype(jnp.int32)
                    plsc.addupdate_scatter(hist_v, [laneoff + bn], ones16)

            return c

        lax.fori_loop(0, CHUNKS // 2, hist_grp, 0)

        carry = jnp.float32(0.0)
        for j in range(16):
            v = hist_v[pl.ds(16 * j, 16)]
            for l in range(1, 16):
                v = v + hist_v[pl.ds(l * 256 + 16 * j, 16)]
            c = plsc.cumsum(v) + carry
            cum_v[pl.ds(16 * j, 16)] = c
            carry = c[15]
        total = carry

        sml = jnp.float32(0.0)
        for j in range(16):
            c = cum_v[pl.ds(16 * j, 16)]
            sml = jnp.maximum(sml, jnp.max(jnp.where(c < total, c, 0.0)))

        def ffloor(x):
            return x.astype(jnp.int32).astype(jnp.float32)

        step = ffloor(jnp.full((16,), sml, jnp.float32) / 255.0)
        off = ffloor(step * 0.5)
        den = jnp.maximum(step, 1.0)
        is_id = step == 0.0

        lut_v[pl.ds(0, 16)] = zeros16
        for j in range(16):
            c = cum_v[pl.ds(16 * j, 16)]
            vals = jnp.clip(ffloor((c + off) / den), 0.0, 255.0)
            idf = lanes_f + jnp.float32(16 * j + 1)
            vals = jnp.where(is_id, idf, vals)
            plsc.store_scatter(lut_v, [lanes + (16 * j + 1)], vals)

        pltpu.async_copy(x_hbm.at[pl.ds(base, K)], in0, sem0)

        def remap_grp(g, c):
            idx0 = g * 2
            for b in range(2):
                inb, semb = ins[b]
                outb, osemb = outs[b]
                idx = idx0 + b
                pltpu.make_async_copy(x_hbm.at[pl.ds(base, K)], inb, semb).wait()
                nxt = idx + 1
                othb, othsem = ins[1 - b]

                @pl.when(nxt < CHUNKS)
                def _():
                    pltpu.async_copy(
                        x_hbm.at[pl.ds(base + nxt * K, K)], othb, othsem)

                @pl.when(g > 0)
                def _():
                    pltpu.make_async_copy(
                        outb, out_hbm.at[pl.ds(base, K)], osemb).wait()

                @plsc.parallel_loop(0, K, step=16, unroll=U)
                def _(o):
                    x = inb[pl.ds(o, 16)]
                    bn = x.astype(jnp.int32)
                    outb[pl.ds(o, 16)] = plsc.load_gather(lut_v, [bn])

                pltpu.async_copy(
                    outb, out_hbm.at[pl.ds(base + idx * K, K)], osemb)
            return c

        lax.fori_loop(0, CHUNKS // 2, remap_grp, 0)
        pltpu.make_async_copy(out0, out_hbm.at[pl.ds(base, K)], osem0).wait()
        pltpu.make_async_copy(out1, out_hbm.at[pl.ds(base, K)], osem1).wait()
        return carry_unused

    lax.fori_loop(0, CPW, per_channel, 0)


def kernel(pic):
    B, C, H, W = pic.shape
    flat = pic.reshape(B * C * H * W)
    mesh = plsc.VectorSubcoreMesh(core_axis_name="c", subcore_axis_name="s")
    f = pl.kernel(
        _histeq_body,
        out_type=jax.ShapeDtypeStruct((NCH * NPX,), jnp.float32),
        mesh=mesh,
        compiler_params=pltpu.CompilerParams(needs_layout_passes=False),
        scratch_types=[
            pltpu.VMEM((K,), jnp.float32),
            pltpu.VMEM((K,), jnp.float32),
            pltpu.VMEM((K,), jnp.float32),
            pltpu.VMEM((K,), jnp.float32),
            pltpu.VMEM((4096,), jnp.float32),
            pltpu.VMEM((256,), jnp.float32),
            pltpu.VMEM((272,), jnp.float32),
            pltpu.SemaphoreType.DMA,
            pltpu.SemaphoreType.DMA,
            pltpu.SemaphoreType.DMA,
            pltpu.SemaphoreType.DMA,
        ],
    )
    out = f(flat)
    return out.reshape(B, C, H, W)
